# Initial kernel scaffold; baseline (speedup 1.0000x reference)
import jax
import jax.numpy as jnp
from jax.experimental import pallas as pl


def kernel(x, edge_index, W1, a_src1, a_dst1, b1, W2, a_src2, a_dst2, b2):
    raise NotImplementedError("write your pallas kernel here")


# speedup vs baseline: 16.6349x; 16.6349x over previous
import functools

import jax
import jax.numpy as jnp
from jax import lax
from jax.experimental import pallas as pl
from jax.experimental.pallas import tpu as pltpu
from jax.experimental.pallas import tpu_sc as plsc

N = 10000
D = 128
WP = 144
NC = 2
NS = 16
NW = NC * NS
L = 16


def _pre_body(h_ref, w_ref, as_ref, ad_ref, xwp_ref, aux_ref):
    xw = jnp.dot(h_ref[...], w_ref[...], preferred_element_type=jnp.float32)
    asr = jnp.sum(xw * as_ref[0:1, :], axis=1, keepdims=True)
    adr = jnp.sum(xw * ad_ref[0:1, :], axis=1, keepdims=True)
    r = xw.shape[0]
    xwp_ref[:, 0:D] = xw
    xwp_ref[:, D:D + 1] = jnp.ones((r, 1), jnp.float32)
    xwp_ref[:, D + 1:WP] = jnp.zeros((r, WP - D - 1), jnp.float32)
    aux_ref[:, 0:1] = asr
    aux_ref[:, 1:2] = adr
    aux_ref[:, 2:16] = jnp.zeros((r, 14), jnp.float32)


def _tc_pre(h, w, asv, adv):
    r = 2000
    grid = (N // r,)
    return pl.pallas_call(
        _pre_body,
        grid=grid,
        in_specs=[
            pl.BlockSpec((r, D), lambda i: (i, 0)),
            pl.BlockSpec((D, D), lambda i: (0, 0)),
            pl.BlockSpec((8, D), lambda i: (0, 0)),
            pl.BlockSpec((8, D), lambda i: (0, 0)),
        ],
        out_specs=[
            pl.BlockSpec((r, WP), lambda i: (i, 0)),
            pl.BlockSpec((r, 16), lambda i: (i, 0)),
        ],
        out_shape=[
            jax.ShapeDtypeStruct((N, WP), jnp.float32),
            jax.ShapeDtypeStruct((N, 16), jnp.float32),
        ],
    )(h, w, asv, adv)


def _post_body(a0_ref, a1_ref, xwp_ref, aux_ref, b_ref, o_ref, *, relu):
    el_s = aux_ref[:, 0:1] + aux_ref[:, 1:2]
    el = jnp.exp(jnp.maximum(el_s, 0.2 * el_s))
    num = a0_ref[:, 0:D] + a1_ref[:, 0:D] + el * xwp_ref[:, 0:D]
    den = a0_ref[:, D:D + 1] + a1_ref[:, D:D + 1] + el + 1e-16
    h = num / den + b_ref[0:1, :]
    if relu:
        h = jnp.maximum(h, 0.0)
    o_ref[...] = h


def _tc_post(a0, a1, xwp, aux, bv, relu):
    r = 2000
    grid = (N // r,)
    return pl.pallas_call(
        functools.partial(_post_body, relu=relu),
        grid=grid,
        in_specs=[
            pl.BlockSpec((r, WP), lambda i: (i, 0)),
            pl.BlockSpec((r, WP), lambda i: (i, 0)),
            pl.BlockSpec((r, WP), lambda i: (i, 0)),
            pl.BlockSpec((r, 16), lambda i: (i, 0)),
            pl.BlockSpec((8, D), lambda i: (0, 0)),
        ],
        out_specs=pl.BlockSpec((r, D), lambda i: (i, 0)),
        out_shape=jax.ShapeDtypeStruct((N, D), jnp.float32),
    )(a0, a1, xwp, aux, bv)


@functools.lru_cache(maxsize=None)
def _mesh():
    return plsc.VectorSubcoreMesh(
        core_axis_name="c", subcore_axis_name="s",
        num_cores=NC, num_subcores=NS)


def _sc_edge(xwp, aux, src_idx, dst_idx):
    e = src_idx.shape[0]
    ept = e // NW
    chunk = 80
    nchunk = ept // chunk
    ngrp = chunk // L
    rows_per_tile = N // NS

    @functools.partial(
        pl.kernel,
        out_type=jax.ShapeDtypeStruct((NC, N, WP), jnp.float32),
        mesh=_mesh(),
        compiler_params=pltpu.CompilerParams(
            use_tc_tiling_on_sc=False, needs_layout_passes=False),
        scratch_types=[
            pltpu.VMEM((1, chunk), jnp.int32),
            pltpu.VMEM((1, chunk), jnp.int32),
            pltpu.VMEM((chunk, 16), jnp.float32),
            pltpu.VMEM((chunk, 16), jnp.float32),
            pltpu.VMEM((chunk, WP), jnp.float32),
            pltpu.VMEM_SHARED((N, WP), jnp.float32),
        ],
    )
    def edge_kernel(xwp_hbm, aux_hbm, si_hbm, di_hbm, out_hbm,
                    srcc_v, dstc_v, auxs_v, auxd_v, rows_v, acc_sh):
        cid = lax.axis_index("c")
        sid = lax.axis_index("s")
        wid = cid * NS + sid
        eb = wid * ept

        z16 = jnp.zeros((L,), jnp.float32)

        @pl.loop(0, chunk)
        def _(r):
            for q in range(WP // L):
                rows_v[r, pl.ds(q * L, L)] = z16

        row0 = sid * rows_per_tile
        nfull, rem = rows_per_tile // chunk, rows_per_tile % chunk
        for i in range(nfull):
            pltpu.sync_copy(rows_v, acc_sh.at[pl.ds(row0 + i * chunk, chunk)])
        if rem:
            pltpu.sync_copy(rows_v.at[pl.ds(0, rem)],
                            acc_sh.at[pl.ds(row0 + nfull * chunk, rem)])
        plsc.subcore_barrier()

        zero16 = jnp.zeros((L,), jnp.int32)
        one16 = jnp.ones((L,), jnp.int32)
        iota16 = jax.lax.iota(jnp.int32, L)

        @pl.loop(0, nchunk)
        def _(c):
            base = eb + c * chunk
            pltpu.sync_copy(si_hbm.at[pl.ds(base, chunk)], srcc_v.at[0])
            pltpu.sync_copy(di_hbm.at[pl.ds(base, chunk)], dstc_v.at[0])
            pltpu.sync_copy(xwp_hbm.at[srcc_v.at[0]], rows_v)
            pltpu.sync_copy(aux_hbm.at[srcc_v.at[0]], auxs_v)
            pltpu.sync_copy(aux_hbm.at[dstc_v.at[0]], auxd_v)

            @pl.loop(0, ngrp)
            def _(g):
                idx16 = iota16 + g * L
                a_s = plsc.load_gather(auxs_v, [idx16, zero16])
                a_d = plsc.load_gather(auxd_v, [idx16, one16])
                ez = a_s + a_d
                ez = jnp.maximum(ez, 0.2 * ez)
                ee = jnp.exp(ez)
                for j in range(L):
                    ev = lax.gather(
                        ee, jnp.full((L, 1), j, jnp.int32),
                        lax.GatherDimensionNumbers(
                            offset_dims=(), collapsed_slice_dims=(0,),
                            start_index_map=(0,)),
                        (1,), mode=lax.GatherScatterMode.PROMISE_IN_BOUNDS)
                    r = g * L + j
                    for q in range(WP // L):
                        sl = pl.ds(q * L, L)
                        rows_v[r, sl] = rows_v[r, sl] * ev

            pltpu.sync_copy(rows_v, acc_sh.at[dstc_v.at[0]], add=True)

        plsc.subcore_barrier()
        pltpu.sync_copy(acc_sh.at[pl.ds(row0, rows_per_tile)],
                        out_hbm.at[cid, pl.ds(row0, rows_per_tile)])

    return edge_kernel(xwp, aux, src_idx, dst_idx)


def _row8(v):
    return jnp.zeros((8, D), jnp.float32).at[0].set(v.reshape(-1))


def kernel(x, edge_index, W1, a_src1, a_dst1, b1, W2, a_src2, a_dst2, b2):
    asv1, adv1 = _row8(a_src1), _row8(a_dst1)
    asv2, adv2 = _row8(a_src2), _row8(a_dst2)
    bv1, bv2 = _row8(b1), _row8(b2)

    src_idx, dst_idx = edge_index[0], edge_index[1]

    xwp1, aux1 = _tc_pre(x, W1, asv1, adv1)
    acc = _sc_edge(xwp1, aux1, src_idx, dst_idx)
    h = _tc_post(acc[0], acc[1], xwp1, aux1, bv1, True)

    xwp2, aux2 = _tc_pre(h, W2, asv2, adv2)
    acc2 = _sc_edge(xwp2, aux2, src_idx, dst_idx)
    return _tc_post(acc2[0], acc2[1], xwp2, aux2, bv2, False)
